# baseline (speedup 1.0000x reference)
import functools

import jax
import jax.numpy as jnp
from jax import lax
from jax.experimental import pallas as pl
from jax.experimental.pallas import tpu as pltpu
from jax.experimental.pallas import tpu_sc as plsc

N = 10000
E = 320000
IN_CH = 128
HID = 256
OUT_CH = 128

NC = 2
NS = 16
CHUNK = 128

E_PAD = 327680
PER_W = E_PAD // (NC * NS)
CHUNKS_W = PER_W // CHUNK
HALF_W = CHUNKS_W // 2
ACC_ROWS = 10112
ROWS_PER_S = ACC_ROWS // NS


def _mesh():
    return plsc.VectorSubcoreMesh(
        core_axis_name="c", subcore_axis_name="s",
        num_cores=NC, num_subcores=NS)


_STACKED_OUT = jax.ShapeDtypeStruct((NC, ACC_ROWS, 128), jnp.float32)


def _zero_accum(z_hbm, accum, s):
    pltpu.sync_copy(
        z_hbm.at[pl.ds(s * ROWS_PER_S, ROWS_PER_S)],
        accum.at[pl.ds(s * ROWS_PER_S, ROWS_PER_S)],
    )


def _write_out(accum, out_hbm, c, s):
    pltpu.sync_copy(
        accum.at[pl.ds(s * ROWS_PER_S, ROWS_PER_S)],
        out_hbm.at[c, pl.ds(s * ROWS_PER_S, ROWS_PER_S)],
    )


@functools.lru_cache(maxsize=None)
def _deg_kernel():
    return functools.partial(
        pl.kernel,
        out_type=_STACKED_OUT,
        mesh=_mesh(),
        scratch_types=[
            pltpu.VMEM((CHUNKS_W, CHUNK), jnp.int32),
            pltpu.VMEM((CHUNK, 128), jnp.float32),
            pltpu.MemorySpace.VMEM_SHARED((ACC_ROWS, 128), jnp.float32),
            pltpu.SemaphoreType.DMA,
        ],
    )(_deg_body)


def _deg_body(dst2_hbm, z_hbm, ones_hbm, out_hbm, didx_all, ones_v, accum, sem):
    c = lax.axis_index("c")
    s = lax.axis_index("s")
    wid = s * NC + c
    row0 = wid * CHUNKS_W
    pltpu.sync_copy(dst2_hbm.at[pl.ds(row0, CHUNKS_W)], didx_all)
    _zero_accum(z_hbm, accum, s)
    pltpu.sync_copy(ones_hbm, ones_v)
    plsc.subcore_barrier()

    def fire(i, carry):
        pltpu.async_copy(ones_v, accum.at[didx_all.at[i]], sem, add=True)
        return carry

    def drain(i, carry):
        pltpu.make_async_copy(ones_v, accum.at[didx_all.at[i]], sem).wait()
        return carry

    lax.fori_loop(0, CHUNKS_W, fire, 0)
    lax.fori_loop(0, CHUNKS_W, drain, 0)
    plsc.subcore_barrier()
    _write_out(accum, out_hbm, c, s)


@functools.lru_cache(maxsize=None)
def _msg_kernel():
    return functools.partial(
        pl.kernel,
        out_type=_STACKED_OUT,
        mesh=_mesh(),
        scratch_types=[
            pltpu.VMEM((CHUNKS_W // 2, CHUNK), jnp.int32),
            pltpu.VMEM((CHUNKS_W // 2, CHUNK), jnp.int32),
            pltpu.VMEM((CHUNK, 128), jnp.float32),
            pltpu.VMEM((CHUNK, 128), jnp.float32),
            pltpu.MemorySpace.VMEM_SHARED((ACC_ROWS, 128), jnp.float32),
            pltpu.SemaphoreType.DMA,
            pltpu.SemaphoreType.DMA,
            pltpu.SemaphoreType.DMA,
            pltpu.SemaphoreType.DMA,
        ],
    )(_msg_body)


def _msg_body(table_hbm, src2_hbm, dst2_hbm, z_hbm, out_hbm,
              sidx_all, didx_all, rows_a, rows_b, accum,
              gsem_a, gsem_b, ssem_a, ssem_b):
    c = lax.axis_index("c")
    s = lax.axis_index("s")
    wid = s * NC + c
    row0 = wid * CHUNKS_W
    half_chunks = CHUNKS_W // 2
    _zero_accum(z_hbm, accum, s)
    plsc.subcore_barrier()

    def start_gather(i, buf, sem):
        pltpu.async_copy(table_hbm.at[sidx_all.at[i]], buf, sem)

    def wait_gather(i, buf, sem):
        pltpu.make_async_copy(table_hbm.at[sidx_all.at[i]], buf, sem).wait()

    def start_scatter(i, buf, sem):
        pltpu.async_copy(buf, accum.at[didx_all.at[i]], sem, add=True)

    def wait_scatter(i, buf, sem):
        pltpu.make_async_copy(buf, accum.at[didx_all.at[i]], sem).wait()

    for h in range(2):
        base_row = row0 + h * half_chunks
        pltpu.sync_copy(src2_hbm.at[pl.ds(base_row, half_chunks)], sidx_all)
        pltpu.sync_copy(dst2_hbm.at[pl.ds(base_row, half_chunks)], didx_all)
        start_gather(0, rows_a, gsem_a)
        start_gather(1, rows_b, gsem_b)

        def body(j, carry):
            i0 = 2 * j
            wait_gather(i0, rows_a, gsem_a)
            start_scatter(i0, rows_a, ssem_a)
            wait_gather(i0 + 1, rows_b, gsem_b)
            start_scatter(i0 + 1, rows_b, ssem_b)
            wait_scatter(i0, rows_a, ssem_a)
            start_gather(i0 + 2, rows_a, gsem_a)
            wait_scatter(i0 + 1, rows_b, ssem_b)
            start_gather(i0 + 3, rows_b, gsem_b)
            return carry

        lax.fori_loop(0, half_chunks // 2 - 1, body, 0)
        i0 = half_chunks - 2
        wait_gather(i0, rows_a, gsem_a)
        start_scatter(i0, rows_a, ssem_a)
        wait_gather(i0 + 1, rows_b, gsem_b)
        start_scatter(i0 + 1, rows_b, ssem_b)
        wait_scatter(i0, rows_a, ssem_a)
        wait_scatter(i0 + 1, rows_b, ssem_b)

    plsc.subcore_barrier()
    _write_out(accum, out_hbm, c, s)


ROWB = 1000


def _dinv(d_ref):
    deg = d_ref[0, :, 0:1] + d_ref[1, :, 0:1] + 1.0
    return lax.rsqrt(deg)


def _mm1_body(x_ref, w_ref, d_ref, ga_ref, gb_ref):
    dinv = _dinv(d_ref)
    h = jnp.dot(x_ref[...], w_ref[...], preferred_element_type=jnp.float32)
    g = h * dinv
    ga_ref[...] = g[:, :128]
    gb_ref[...] = g[:, 128:]


def _mm2_body(ma_ref, mb_ref, ga_ref, gb_ref, d_ref, b1_ref, w2_ref, out_ref):
    dinv = _dinv(d_ref)
    m0 = ma_ref[0] + ma_ref[1] + ga_ref[...]
    m1 = mb_ref[0] + mb_ref[1] + gb_ref[...]
    s = jnp.concatenate([m0, m1], axis=1)
    out1 = jnp.maximum(s * dinv + b1_ref[...], 0.0)
    h2 = jnp.dot(out1, w2_ref[...], preferred_element_type=jnp.float32)
    out_ref[...] = h2 * dinv


def _fin_body(q_ref, g2_ref, d_ref, b2_ref, out_ref):
    dinv = _dinv(d_ref)
    s = q_ref[0] + q_ref[1] + g2_ref[...]
    out_ref[...] = jnp.maximum(s * dinv + b2_ref[...], 0.0)


def _rows_spec(w):
    return pl.BlockSpec((ROWB, w), lambda r: (r, 0))


def _stk_spec():
    return pl.BlockSpec((NC, ROWB, 128), lambda r: (0, r, 0))


def _full_spec(h, w):
    return pl.BlockSpec((h, w), lambda r: (0, 0))


def kernel(x, edge_index, W1, b1, W2, b2):
    src = edge_index[0].astype(jnp.int32)
    dst = edge_index[1].astype(jnp.int32)
    pad = E_PAD - E
    src_p = jnp.concatenate([src, jnp.zeros((pad,), jnp.int32)]).reshape(
        E_PAD // CHUNK, CHUNK)
    dst_p = jnp.concatenate([dst, jnp.full((pad,), N, jnp.int32)]).reshape(
        E_PAD // CHUNK, CHUNK)
    z128 = jnp.zeros((ACC_ROWS, 128), jnp.float32)
    ones128 = jnp.ones((CHUNK, 128), jnp.float32)
    b1r = b1.reshape(1, HID)
    b2r = b2.reshape(1, OUT_CH)

    degs = _deg_kernel()(dst_p, z128, ones128)

    g1a, g1b = pl.pallas_call(
        _mm1_body,
        grid=(N // ROWB,),
        in_specs=[
            _rows_spec(IN_CH),
            _full_spec(IN_CH, HID),
            _stk_spec(),
        ],
        out_specs=[_rows_spec(128), _rows_spec(128)],
        out_shape=[
            jax.ShapeDtypeStruct((N, 128), jnp.float32),
            jax.ShapeDtypeStruct((N, 128), jnp.float32),
        ],
    )(x, W1, degs)

    ma = _msg_kernel()(g1a, src_p, dst_p, z128)
    mb = _msg_kernel()(g1b, src_p, dst_p, z128)

    g2 = pl.pallas_call(
        _mm2_body,
        grid=(N // ROWB,),
        in_specs=[
            _stk_spec(),
            _stk_spec(),
            _rows_spec(128),
            _rows_spec(128),
            _stk_spec(),
            _full_spec(1, HID),
            _full_spec(HID, OUT_CH),
        ],
        out_specs=_rows_spec(OUT_CH),
        out_shape=jax.ShapeDtypeStruct((N, OUT_CH), jnp.float32),
    )(ma, mb, g1a, g1b, degs, b1r, W2)

    qs = _msg_kernel()(g2, src_p, dst_p, z128)

    out = pl.pallas_call(
        _fin_body,
        grid=(N // ROWB,),
        in_specs=[
            _stk_spec(),
            _rows_spec(OUT_CH),
            _stk_spec(),
            _full_spec(1, OUT_CH),
        ],
        out_specs=_rows_spec(OUT_CH),
        out_shape=jax.ShapeDtypeStruct((N, OUT_CH), jnp.float32),
    )(qs, g2, degs, b2r)

    return out


# speedup vs baseline: 1.1941x; 1.1941x over previous
import functools

import jax
import jax.numpy as jnp
from jax import lax
from jax.experimental import pallas as pl
from jax.experimental.pallas import tpu as pltpu
from jax.experimental.pallas import tpu_sc as plsc

N = 10000
E = 320000
IN_CH = 128
HID = 256
OUT_CH = 128

NC = 2
NS = 16
CHUNK = 128

E_PAD = 323584
PER_W = E_PAD // (NC * NS)
ACC_ROWS = 10112
ROWS_PER_S = ACC_ROWS // NS


def _mesh():
    return plsc.VectorSubcoreMesh(
        core_axis_name="c", subcore_axis_name="s",
        num_cores=NC, num_subcores=NS)


_STACKED_OUT = jax.ShapeDtypeStruct((NC, ACC_ROWS, 128), jnp.float32)


def _zero_accum(z_hbm, accum, s):
    pltpu.sync_copy(
        z_hbm.at[pl.ds(s * ROWS_PER_S, ROWS_PER_S)],
        accum.at[pl.ds(s * ROWS_PER_S, ROWS_PER_S)],
    )


def _write_out(accum, out_hbm, c, s):
    pltpu.sync_copy(
        accum.at[pl.ds(s * ROWS_PER_S, ROWS_PER_S)],
        out_hbm.at[c, pl.ds(s * ROWS_PER_S, ROWS_PER_S)],
    )


@functools.lru_cache(maxsize=None)
def _deg_kernel():
    return functools.partial(
        pl.kernel,
        out_type=_STACKED_OUT,
        mesh=_mesh(),
        scratch_types=[
            pltpu.VMEM((CHUNK,), jnp.int32),
            pltpu.VMEM((CHUNK, 128), jnp.float32),
            pltpu.MemorySpace.VMEM_SHARED((ACC_ROWS, 128), jnp.float32),
        ],
    )(_deg_body)


def _deg_body(dst_hbm, z_hbm, ones_hbm, out_hbm, didx, ones_v, accum):
    c = lax.axis_index("c")
    s = lax.axis_index("s")
    _zero_accum(z_hbm, accum, s)
    pltpu.sync_copy(ones_hbm, ones_v)
    plsc.subcore_barrier()
    base = (s * NC + c) * PER_W

    def body(i, carry):
        pltpu.sync_copy(dst_hbm.at[pl.ds(base + i * CHUNK, CHUNK)], didx)
        pltpu.sync_copy(ones_v, accum.at[didx], add=True)
        return carry

    lax.fori_loop(0, PER_W // CHUNK, body, 0)
    plsc.subcore_barrier()
    _write_out(accum, out_hbm, c, s)


@functools.lru_cache(maxsize=None)
def _msg_kernel():
    return functools.partial(
        pl.kernel,
        out_type=_STACKED_OUT,
        mesh=_mesh(),
        scratch_types=[
            pltpu.VMEM((CHUNK,), jnp.int32),
            pltpu.VMEM((CHUNK,), jnp.int32),
            pltpu.VMEM((CHUNK, 128), jnp.float32),
            pltpu.MemorySpace.VMEM_SHARED((ACC_ROWS, 128), jnp.float32),
            pltpu.SemaphoreType.DMA,
        ],
    )(_msg_body)


def _msg_body(table_hbm, src_hbm, dst_hbm, z_hbm, out_hbm,
              sidx, didx, rows, accum, sem):
    c = lax.axis_index("c")
    s = lax.axis_index("s")
    _zero_accum(z_hbm, accum, s)
    plsc.subcore_barrier()
    base = (s * NC + c) * PER_W

    def body(i, carry):
        off = base + i * CHUNK
        pltpu.sync_copy(src_hbm.at[pl.ds(off, CHUNK)], sidx)
        pltpu.sync_copy(dst_hbm.at[pl.ds(off, CHUNK)], didx)
        pltpu.async_copy(table_hbm.at[sidx], rows, sem).wait()
        pltpu.sync_copy(rows, accum.at[didx], add=True)
        return carry

    lax.fori_loop(0, PER_W // CHUNK, body, 0)
    plsc.subcore_barrier()
    _write_out(accum, out_hbm, c, s)


ROWB = 1000


def _dinv(d_ref):
    deg = d_ref[0, :, 0:1] + d_ref[1, :, 0:1] + 1.0
    return lax.rsqrt(deg)


def _mm1_body(x_ref, w_ref, d_ref, ga_ref, gb_ref):
    dinv = _dinv(d_ref)
    h = jnp.dot(x_ref[...], w_ref[...], preferred_element_type=jnp.float32)
    g = h * dinv
    ga_ref[...] = g[:, :128]
    gb_ref[...] = g[:, 128:]


def _mm2_body(ma_ref, mb_ref, ga_ref, gb_ref, d_ref, b1_ref, w2_ref, out_ref):
    dinv = _dinv(d_ref)
    m0 = ma_ref[0] + ma_ref[1] + ga_ref[...]
    m1 = mb_ref[0] + mb_ref[1] + gb_ref[...]
    s = jnp.concatenate([m0, m1], axis=1)
    out1 = jnp.maximum(s * dinv + b1_ref[...], 0.0)
    h2 = jnp.dot(out1, w2_ref[...], preferred_element_type=jnp.float32)
    out_ref[...] = h2 * dinv


def _fin_body(q_ref, g2_ref, d_ref, b2_ref, out_ref):
    dinv = _dinv(d_ref)
    s = q_ref[0] + q_ref[1] + g2_ref[...]
    out_ref[...] = jnp.maximum(s * dinv + b2_ref[...], 0.0)


def _rows_spec(w):
    return pl.BlockSpec((ROWB, w), lambda r: (r, 0))


def _stk_spec():
    return pl.BlockSpec((NC, ROWB, 128), lambda r: (0, r, 0))


def _full_spec(h, w):
    return pl.BlockSpec((h, w), lambda r: (0, 0))


def kernel(x, edge_index, W1, b1, W2, b2):
    src = edge_index[0].astype(jnp.int32)
    dst = edge_index[1].astype(jnp.int32)
    pad = E_PAD - E
    src_p = jnp.concatenate([src, jnp.zeros((pad,), jnp.int32)])
    dst_p = jnp.concatenate([dst, jnp.full((pad,), N, jnp.int32)])
    z128 = jnp.zeros((ACC_ROWS, 128), jnp.float32)
    ones128 = jnp.ones((CHUNK, 128), jnp.float32)
    b1r = b1.reshape(1, HID)
    b2r = b2.reshape(1, OUT_CH)

    degs = _deg_kernel()(dst_p, z128, ones128)

    g1a, g1b = pl.pallas_call(
        _mm1_body,
        grid=(N // ROWB,),
        in_specs=[
            _rows_spec(IN_CH),
            _full_spec(IN_CH, HID),
            _stk_spec(),
        ],
        out_specs=[_rows_spec(128), _rows_spec(128)],
        out_shape=[
            jax.ShapeDtypeStruct((N, 128), jnp.float32),
            jax.ShapeDtypeStruct((N, 128), jnp.float32),
        ],
    )(x, W1, degs)

    ma = _msg_kernel()(g1a, src_p, dst_p, z128)
    mb = _msg_kernel()(g1b, src_p, dst_p, z128)

    g2 = pl.pallas_call(
        _mm2_body,
        grid=(N // ROWB,),
        in_specs=[
            _stk_spec(),
            _stk_spec(),
            _rows_spec(128),
            _rows_spec(128),
            _stk_spec(),
            _full_spec(1, HID),
            _full_spec(HID, OUT_CH),
        ],
        out_specs=_rows_spec(OUT_CH),
        out_shape=jax.ShapeDtypeStruct((N, OUT_CH), jnp.float32),
    )(ma, mb, g1a, g1b, degs, b1r, W2)

    qs = _msg_kernel()(g2, src_p, dst_p, z128)

    out = pl.pallas_call(
        _fin_body,
        grid=(N // ROWB,),
        in_specs=[
            _stk_spec(),
            _rows_spec(OUT_CH),
            _stk_spec(),
            _full_spec(1, OUT_CH),
        ],
        out_specs=_rows_spec(OUT_CH),
        out_shape=jax.ShapeDtypeStruct((N, OUT_CH), jnp.float32),
    )(qs, g2, degs, b2r)

    return out
